# baseline (speedup 1.0000x reference)
import jax
import jax.numpy as jnp
from jax.experimental import pallas as pl
from jax.experimental.pallas import tpu as pltpu

_GLOBAL_WEIGHT = 1.0
_LOCAL_WEIGHT = 2.0

_ROWS = 4096
_COLS = 1024
_BLOCK_ROWS = 64


def _body(p_ref, o_ref, gm_ref, lm_ref, out_ref):
    i = pl.program_id(0)

    d = p_ref[...] - o_ref[...]
    d2 = d * d
    lm = lm_ref[...] != 0
    gm_only = jnp.logical_and(gm_ref[...] != 0, jnp.logical_not(lm))

    gs = jnp.sum(jnp.where(gm_only, d2, 0.0))
    gc = jnp.sum(gm_only.astype(jnp.float32))
    ls = jnp.sum(jnp.where(lm, d2, 0.0))
    lc = jnp.sum(lm.astype(jnp.float32))

    @pl.when(i == 0)
    def _init():
        out_ref[0] = 0.0
        out_ref[1] = 0.0
        out_ref[2] = 0.0
        out_ref[3] = 0.0

    out_ref[0] += gs
    out_ref[1] += gc
    out_ref[2] += ls
    out_ref[3] += lc


def kernel(predicted_image, original_image, global_mask, local_mask):
    p = predicted_image.reshape(_ROWS, _COLS)
    o = original_image.reshape(_ROWS, _COLS)
    gm = global_mask.view(jnp.int8).reshape(_ROWS, _COLS)
    lm = local_mask.view(jnp.int8).reshape(_ROWS, _COLS)

    grid = (_ROWS // _BLOCK_ROWS,)
    in_spec = pl.BlockSpec((_BLOCK_ROWS, _COLS), lambda i: (i, 0))

    sums = pl.pallas_call(
        _body,
        grid=grid,
        in_specs=[in_spec, in_spec, in_spec, in_spec],
        out_specs=pl.BlockSpec(memory_space=pltpu.SMEM),
        out_shape=jax.ShapeDtypeStruct((4,), jnp.float32),
    )(p, o, gm, lm)

    global_loss = sums[0] / (sums[1] + 1e-08)
    local_loss = sums[2] / (sums[3] + 1e-08)
    total_loss = _GLOBAL_WEIGHT * global_loss + _LOCAL_WEIGHT * local_loss
    return (total_loss, global_loss, local_loss)


# speedup vs baseline: 1.2845x; 1.2845x over previous
import jax
import jax.numpy as jnp
from jax.experimental import pallas as pl
from jax.experimental.pallas import tpu as pltpu

_GLOBAL_WEIGHT = 1.0
_LOCAL_WEIGHT = 2.0

_ROWS = 4096
_COLS = 1024
_BLOCK_ROWS = 256


def _body(p_ref, o_ref, gm_ref, lm_ref, out_ref, acc_ref):
    i = pl.program_id(0)

    @pl.when(i == 0)
    def _init():
        acc_ref[...] = jnp.zeros_like(acc_ref)

    d = p_ref[...] - o_ref[...]
    d2 = d * d
    lmf = lm_ref[...].astype(jnp.float32)
    gmf = gm_ref[...].astype(jnp.float32)
    gof = gmf * (1.0 - lmf)

    r = _BLOCK_ROWS // 8
    acc_ref[0] += jnp.sum((d2 * gof).reshape(r, 8, _COLS), axis=0)
    acc_ref[1] += jnp.sum(gof.reshape(r, 8, _COLS), axis=0)
    acc_ref[2] += jnp.sum((d2 * lmf).reshape(r, 8, _COLS), axis=0)
    acc_ref[3] += jnp.sum(lmf.reshape(r, 8, _COLS), axis=0)

    @pl.when(i == pl.num_programs(0) - 1)
    def _finish():
        out_ref[0] = jnp.sum(acc_ref[0])
        out_ref[1] = jnp.sum(acc_ref[1])
        out_ref[2] = jnp.sum(acc_ref[2])
        out_ref[3] = jnp.sum(acc_ref[3])


def kernel(predicted_image, original_image, global_mask, local_mask):
    p = predicted_image.reshape(_ROWS, _COLS)
    o = original_image.reshape(_ROWS, _COLS)
    gm = global_mask.view(jnp.int8).reshape(_ROWS, _COLS)
    lm = local_mask.view(jnp.int8).reshape(_ROWS, _COLS)

    grid = (_ROWS // _BLOCK_ROWS,)
    in_spec = pl.BlockSpec((_BLOCK_ROWS, _COLS), lambda i: (i, 0))

    sums = pl.pallas_call(
        _body,
        grid=grid,
        in_specs=[in_spec, in_spec, in_spec, in_spec],
        out_specs=pl.BlockSpec(memory_space=pltpu.SMEM),
        out_shape=jax.ShapeDtypeStruct((4,), jnp.float32),
        scratch_shapes=[pltpu.VMEM((4, 8, _COLS), jnp.float32)],
    )(p, o, gm, lm)

    global_loss = sums[0] / (sums[1] + 1e-08)
    local_loss = sums[2] / (sums[3] + 1e-08)
    total_loss = _GLOBAL_WEIGHT * global_loss + _LOCAL_WEIGHT * local_loss
    return (total_loss, global_loss, local_loss)


# speedup vs baseline: 1.3944x; 1.0856x over previous
import jax
import jax.numpy as jnp
from jax.experimental import pallas as pl
from jax.experimental.pallas import tpu as pltpu

_GLOBAL_WEIGHT = 1.0
_LOCAL_WEIGHT = 2.0

_ROWS = 4096
_COLS = 1024
_CHUNK_ROWS = 128
_NCHUNKS = _ROWS // _CHUNK_ROWS
_NBUF = 8


def _body(p_hbm, o_hbm, gm_hbm, lm_hbm, out_ref,
          p_buf, o_buf, gm_buf, lm_buf, acc_ref,
          p_sem, o_sem, gm_sem, lm_sem):

    def copies(c, s):
        row = c * _CHUNK_ROWS
        return (
            pltpu.make_async_copy(
                p_hbm.at[pl.ds(row, _CHUNK_ROWS), :], p_buf.at[s], p_sem.at[s]),
            pltpu.make_async_copy(
                o_hbm.at[pl.ds(row, _CHUNK_ROWS), :], o_buf.at[s], o_sem.at[s]),
            pltpu.make_async_copy(
                gm_hbm.at[pl.ds(row, _CHUNK_ROWS), :], gm_buf.at[s], gm_sem.at[s]),
            pltpu.make_async_copy(
                lm_hbm.at[pl.ds(row, _CHUNK_ROWS), :], lm_buf.at[s], lm_sem.at[s]),
        )

    def start(c, s):
        for cp in copies(c, s):
            cp.start()

    acc_ref[...] = jnp.zeros_like(acc_ref)

    for c in range(_NBUF):
        start(c, c)

    def loop_body(c, _):
        s = jax.lax.rem(c, _NBUF)
        for cp in copies(c, s):
            cp.wait()

        d = p_buf[s] - o_buf[s]
        d2 = d * d
        lmf = lm_buf[s].astype(jnp.float32)
        gmf = gm_buf[s].astype(jnp.float32)
        gof = gmf * (1.0 - lmf)

        r = _CHUNK_ROWS // 8
        acc_ref[0] += jnp.sum((d2 * gof).reshape(r, 8, _COLS), axis=0)
        acc_ref[1] += jnp.sum(gof.reshape(r, 8, _COLS), axis=0)
        acc_ref[2] += jnp.sum((d2 * lmf).reshape(r, 8, _COLS), axis=0)
        acc_ref[3] += jnp.sum(lmf.reshape(r, 8, _COLS), axis=0)

        @pl.when(c + _NBUF < _NCHUNKS)
        def _next():
            start(c + _NBUF, s)

        return 0

    jax.lax.fori_loop(0, _NCHUNKS, loop_body, 0)

    out_ref[0] = jnp.sum(acc_ref[0])
    out_ref[1] = jnp.sum(acc_ref[1])
    out_ref[2] = jnp.sum(acc_ref[2])
    out_ref[3] = jnp.sum(acc_ref[3])


def kernel(predicted_image, original_image, global_mask, local_mask):
    p = predicted_image.reshape(_ROWS, _COLS)
    o = original_image.reshape(_ROWS, _COLS)
    gm = global_mask.view(jnp.int8).reshape(_ROWS, _COLS)
    lm = local_mask.view(jnp.int8).reshape(_ROWS, _COLS)

    any_spec = pl.BlockSpec(memory_space=pl.ANY)

    sums = pl.pallas_call(
        _body,
        in_specs=[any_spec, any_spec, any_spec, any_spec],
        out_specs=pl.BlockSpec(memory_space=pltpu.SMEM),
        out_shape=jax.ShapeDtypeStruct((4,), jnp.float32),
        scratch_shapes=[
            pltpu.VMEM((_NBUF, _CHUNK_ROWS, _COLS), jnp.float32),
            pltpu.VMEM((_NBUF, _CHUNK_ROWS, _COLS), jnp.float32),
            pltpu.VMEM((_NBUF, _CHUNK_ROWS, _COLS), jnp.int8),
            pltpu.VMEM((_NBUF, _CHUNK_ROWS, _COLS), jnp.int8),
            pltpu.VMEM((4, 8, _COLS), jnp.float32),
            pltpu.SemaphoreType.DMA((_NBUF,)),
            pltpu.SemaphoreType.DMA((_NBUF,)),
            pltpu.SemaphoreType.DMA((_NBUF,)),
            pltpu.SemaphoreType.DMA((_NBUF,)),
        ],
    )(p, o, gm, lm)

    global_loss = sums[0] / (sums[1] + 1e-08)
    local_loss = sums[2] / (sums[3] + 1e-08)
    total_loss = _GLOBAL_WEIGHT * global_loss + _LOCAL_WEIGHT * local_loss
    return (total_loss, global_loss, local_loss)


# speedup vs baseline: 3.6744x; 2.6351x over previous
import jax
import jax.numpy as jnp
from jax.experimental import pallas as pl
from jax.experimental.pallas import tpu as pltpu

_GLOBAL_WEIGHT = 1.0
_LOCAL_WEIGHT = 2.0

_ROWS = 32768
_COLS = 128
_CHUNK_ROWS = 1024
_NCHUNKS = _ROWS // _CHUNK_ROWS
_NBUF = 8


def _body(p_hbm, o_hbm, gm_hbm, lm_hbm, out_ref,
          p_buf, o_buf, gm_buf, lm_buf, acc_ref,
          p_sem, o_sem, gm_sem, lm_sem):

    def copies(c, s):
        row = c * _CHUNK_ROWS
        return (
            pltpu.make_async_copy(
                p_hbm.at[pl.ds(row, _CHUNK_ROWS), :], p_buf.at[s], p_sem.at[s]),
            pltpu.make_async_copy(
                o_hbm.at[pl.ds(row, _CHUNK_ROWS), :], o_buf.at[s], o_sem.at[s]),
            pltpu.make_async_copy(
                gm_hbm.at[pl.ds(row, _CHUNK_ROWS), :], gm_buf.at[s], gm_sem.at[s]),
            pltpu.make_async_copy(
                lm_hbm.at[pl.ds(row, _CHUNK_ROWS), :], lm_buf.at[s], lm_sem.at[s]),
        )

    def start(c, s):
        for cp in copies(c, s):
            cp.start()

    acc_ref[...] = jnp.zeros_like(acc_ref)

    for c in range(_NBUF):
        start(c, c)

    def loop_body(c, _):
        s = jax.lax.rem(c, _NBUF)
        for cp in copies(c, s):
            cp.wait()

        d = p_buf[s] - o_buf[s]
        d2 = d * d
        lmf = lm_buf[s].astype(jnp.float32)
        gmf = gm_buf[s].astype(jnp.float32)
        gof = gmf * (1.0 - lmf)

        r = _CHUNK_ROWS // 8
        acc_ref[0] += jnp.sum((d2 * gof).reshape(r, 8, _COLS), axis=0)
        acc_ref[1] += jnp.sum(gof.reshape(r, 8, _COLS), axis=0)
        acc_ref[2] += jnp.sum((d2 * lmf).reshape(r, 8, _COLS), axis=0)
        acc_ref[3] += jnp.sum(lmf.reshape(r, 8, _COLS), axis=0)

        @pl.when(c + _NBUF < _NCHUNKS)
        def _next():
            start(c + _NBUF, s)

        return 0

    jax.lax.fori_loop(0, _NCHUNKS, loop_body, 0)

    out_ref[0] = jnp.sum(acc_ref[0])
    out_ref[1] = jnp.sum(acc_ref[1])
    out_ref[2] = jnp.sum(acc_ref[2])
    out_ref[3] = jnp.sum(acc_ref[3])


def kernel(predicted_image, original_image, global_mask, local_mask):
    p = predicted_image.reshape(_ROWS, _COLS)
    o = original_image.reshape(_ROWS, _COLS)
    gm = global_mask.view(jnp.int8).reshape(_ROWS, _COLS)
    lm = local_mask.view(jnp.int8).reshape(_ROWS, _COLS)

    any_spec = pl.BlockSpec(memory_space=pl.ANY)

    sums = pl.pallas_call(
        _body,
        in_specs=[any_spec, any_spec, any_spec, any_spec],
        out_specs=pl.BlockSpec(memory_space=pltpu.SMEM),
        out_shape=jax.ShapeDtypeStruct((4,), jnp.float32),
        scratch_shapes=[
            pltpu.VMEM((_NBUF, _CHUNK_ROWS, _COLS), jnp.float32),
            pltpu.VMEM((_NBUF, _CHUNK_ROWS, _COLS), jnp.float32),
            pltpu.VMEM((_NBUF, _CHUNK_ROWS, _COLS), jnp.int8),
            pltpu.VMEM((_NBUF, _CHUNK_ROWS, _COLS), jnp.int8),
            pltpu.VMEM((4, 8, _COLS), jnp.float32),
            pltpu.SemaphoreType.DMA((_NBUF,)),
            pltpu.SemaphoreType.DMA((_NBUF,)),
            pltpu.SemaphoreType.DMA((_NBUF,)),
            pltpu.SemaphoreType.DMA((_NBUF,)),
        ],
    )(p, o, gm, lm)

    global_loss = sums[0] / (sums[1] + 1e-08)
    local_loss = sums[2] / (sums[3] + 1e-08)
    total_loss = _GLOBAL_WEIGHT * global_loss + _LOCAL_WEIGHT * local_loss
    return (total_loss, global_loss, local_loss)
